# Initial kernel scaffold; baseline (speedup 1.0000x reference)
import jax
import jax.numpy as jnp
from jax.experimental import pallas as pl


def kernel(atom_in_fea, nbr_fea, nbr_fea_idx, ln_scale, ln_bias, W_center, b_center, W_nbr, b_nbr, W_edge, b_edge, W_gate, b_gate, W_mag, b_mag):
    raise NotImplementedError("write your pallas kernel here")


# speedup vs baseline: 1.7523x; 1.7523x over previous
import jax
import jax.numpy as jnp
from jax.experimental import pallas as pl
from jax.experimental.pallas import tpu as pltpu
from jax.experimental.pallas import tpu_sc as plsc


def _proj_body(x_ref, s_ref, b_ref, wc_ref, bc_ref, wn_ref, bn_ref,
               zc_ref, zn_ref):
    x = x_ref[...]
    mu = jnp.mean(x, axis=-1, keepdims=True)
    var = jnp.mean((x - mu) ** 2, axis=-1, keepdims=True)
    normed = (x - mu) * jax.lax.rsqrt(var + 1e-6) * s_ref[...] + b_ref[...]
    zc_ref[...] = jnp.dot(normed, wc_ref[...],
                          preferred_element_type=jnp.float32) + bc_ref[...]
    zn_ref[...] = jnp.dot(normed, wn_ref[...],
                          preferred_element_type=jnp.float32) + bn_ref[...]


def _projections(atom_in_fea, ln_scale, ln_bias, W_center, b_center,
                 W_nbr, b_nbr, block):
    n, d = atom_in_fea.shape
    grid = (n // block,)
    full = lambda i: (0, 0)
    return pl.pallas_call(
        _proj_body,
        grid=grid,
        in_specs=[
            pl.BlockSpec((block, d), lambda i: (i, 0)),
            pl.BlockSpec((1, d), full),
            pl.BlockSpec((1, d), full),
            pl.BlockSpec((d, d), full),
            pl.BlockSpec((1, d), full),
            pl.BlockSpec((d, d), full),
            pl.BlockSpec((1, d), full),
        ],
        out_specs=[
            pl.BlockSpec((block, d), lambda i: (i, 0)),
            pl.BlockSpec((block, d), lambda i: (i, 0)),
        ],
        out_shape=[
            jax.ShapeDtypeStruct((n, d), jnp.float32),
            jax.ShapeDtypeStruct((n, d), jnp.float32),
        ],
    )(atom_in_fea, ln_scale.reshape(1, d), ln_bias.reshape(1, d),
      W_center, b_center.reshape(1, d), W_nbr, b_nbr.reshape(1, d))


def _sc_gather(table, flat_idx, window):
    n, d = table.shape
    num = flat_idx.shape[0]
    idx2 = flat_idx.reshape(1, num)
    mesh = plsc.VectorSubcoreMesh(core_axis_name="core",
                                  subcore_axis_name="subcore")

    @pl.kernel(out_type=jax.ShapeDtypeStruct((num, d), table.dtype),
               mesh=mesh)
    def gather_kernel(x_hbm, i_hbm, o_hbm):
        def body(i_vmem, o_vmem):
            pltpu.sync_copy(x_hbm.at[i_vmem.at[0]], o_vmem)

        pltpu.emit_pipeline(
            body,
            grid=(num // window,),
            in_specs=[pl.BlockSpec((1, window), index_map=lambda i: (0, i))],
            out_specs=[pl.BlockSpec((window, d), index_map=lambda i: (i, 0))],
            core_axis_name=("core", "subcore"),
            dimension_semantics=(pltpu.PARALLEL,),
        )(i_hbm, o_hbm)

    return gather_kernel(table, idx2)


def _inter_body(x_ref, zc_ref, g_ref, nbr_ref, we_ref, be_ref, wg_ref,
                bg_ref, wm_ref, bm_ref, out_ref):
    m = g_ref.shape[1]
    zc = zc_ref[...]
    acc = jnp.zeros_like(zc)
    for j in range(m):
        pe = jnp.dot(nbr_ref[:, j, :], we_ref[...],
                     preferred_element_type=jnp.float32) + be_ref[...]
        inter = zc * g_ref[:, j, :] * pe
        gate = jax.nn.sigmoid(
            jnp.dot(inter, wg_ref[...],
                    preferred_element_type=jnp.float32) + bg_ref[...])
        mag = jax.nn.softplus(
            jnp.dot(inter, wm_ref[...],
                    preferred_element_type=jnp.float32) + bm_ref[...])
        acc = acc + gate * mag
    out_ref[...] = x_ref[...] + acc


def _interaction(atom_in_fea, zc, gathered3, nbr_fea, W_edge, b_edge,
                 W_gate, b_gate, W_mag, b_mag, block):
    n, d = atom_in_fea.shape
    m = nbr_fea.shape[1]
    de = nbr_fea.shape[2]
    grid = (n // block,)
    full = lambda i: (0, 0)
    return pl.pallas_call(
        _inter_body,
        grid=grid,
        in_specs=[
            pl.BlockSpec((block, d), lambda i: (i, 0)),
            pl.BlockSpec((block, d), lambda i: (i, 0)),
            pl.BlockSpec((block, m, d), lambda i: (i, 0, 0)),
            pl.BlockSpec((block, m, de), lambda i: (i, 0, 0)),
            pl.BlockSpec((de, d), full),
            pl.BlockSpec((1, d), full),
            pl.BlockSpec((d, d), full),
            pl.BlockSpec((1, d), full),
            pl.BlockSpec((d, d), full),
            pl.BlockSpec((1, d), full),
        ],
        out_specs=pl.BlockSpec((block, d), lambda i: (i, 0)),
        out_shape=jax.ShapeDtypeStruct((n, d), jnp.float32),
    )(atom_in_fea, zc, gathered3, nbr_fea, W_edge, b_edge.reshape(1, d),
      W_gate, b_gate.reshape(1, d), W_mag, b_mag.reshape(1, d))


def kernel(atom_in_fea, nbr_fea, nbr_fea_idx, ln_scale, ln_bias, W_center,
           b_center, W_nbr, b_nbr, W_edge, b_edge, W_gate, b_gate, W_mag,
           b_mag):
    n, d = atom_in_fea.shape
    m = nbr_fea_idx.shape[1]
    zc, zn = _projections(atom_in_fea, ln_scale, ln_bias, W_center, b_center,
                          W_nbr, b_nbr, block=400)
    gathered = _sc_gather(zn, nbr_fea_idx.reshape(-1), window=128)
    gathered3 = gathered.reshape(n, m, d)
    return _interaction(atom_in_fea, zc, gathered3, nbr_fea, W_edge, b_edge,
                        W_gate, b_gate, W_mag, b_mag, block=200)


# speedup vs baseline: 1.9789x; 1.1293x over previous
import jax
import jax.numpy as jnp
from jax.experimental import pallas as pl
from jax.experimental.pallas import tpu as pltpu
from jax.experimental.pallas import tpu_sc as plsc


def _proj_body(x_ref, s_ref, b_ref, wc_ref, bc_ref, wn_ref, bn_ref,
               zc_ref, zn_ref):
    x = x_ref[...]
    mu = jnp.mean(x, axis=-1, keepdims=True)
    var = jnp.mean((x - mu) ** 2, axis=-1, keepdims=True)
    normed = (x - mu) * jax.lax.rsqrt(var + 1e-6) * s_ref[...] + b_ref[...]
    zc_ref[...] = jnp.dot(normed, wc_ref[...],
                          preferred_element_type=jnp.float32) + bc_ref[...]
    zn_ref[...] = jnp.dot(normed, wn_ref[...],
                          preferred_element_type=jnp.float32) + bn_ref[...]


def _projections(atom_in_fea, ln_scale, ln_bias, W_center, b_center,
                 W_nbr, b_nbr, block):
    n, d = atom_in_fea.shape
    grid = (n // block,)
    full = lambda i: (0, 0)
    return pl.pallas_call(
        _proj_body,
        grid=grid,
        in_specs=[
            pl.BlockSpec((block, d), lambda i: (i, 0)),
            pl.BlockSpec((1, d), full),
            pl.BlockSpec((1, d), full),
            pl.BlockSpec((d, d), full),
            pl.BlockSpec((1, d), full),
            pl.BlockSpec((d, d), full),
            pl.BlockSpec((1, d), full),
        ],
        out_specs=[
            pl.BlockSpec((block, d), lambda i: (i, 0)),
            pl.BlockSpec((block, d), lambda i: (i, 0)),
        ],
        out_shape=[
            jax.ShapeDtypeStruct((n, d), jnp.float32),
            jax.ShapeDtypeStruct((n, d), jnp.float32),
        ],
    )(atom_in_fea, ln_scale.reshape(1, d), ln_bias.reshape(1, d),
      W_center, b_center.reshape(1, d), W_nbr, b_nbr.reshape(1, d))


def _sc_gather(table, flat_idx, window):
    n, d = table.shape
    num = flat_idx.shape[0]
    idx2 = flat_idx.reshape(1, num)
    mesh = plsc.VectorSubcoreMesh(core_axis_name="core",
                                  subcore_axis_name="subcore")

    @pl.kernel(out_type=jax.ShapeDtypeStruct((num, d), table.dtype),
               mesh=mesh)
    def gather_kernel(x_hbm, i_hbm, o_hbm):
        def body(i_vmem, o_vmem):
            pltpu.sync_copy(x_hbm.at[i_vmem.at[0]], o_vmem)

        pltpu.emit_pipeline(
            body,
            grid=(num // window,),
            in_specs=[pl.BlockSpec((1, window), index_map=lambda i: (0, i))],
            out_specs=[pl.BlockSpec((window, d), index_map=lambda i: (i, 0))],
            core_axis_name=("core", "subcore"),
            dimension_semantics=(pltpu.PARALLEL,),
        )(i_hbm, o_hbm)

    return gather_kernel(table, idx2)


def _inter_body(x_ref, zc_ref, g_ref, nbr_ref, we_ref, be_ref, wgm_ref,
                bgm_ref, out_ref):
    m = g_ref.shape[1]
    d = zc_ref.shape[1]
    zc = zc_ref[...]
    acc = jnp.zeros_like(zc)
    for j in range(m):
        pe = jnp.dot(nbr_ref[:, j, :].astype(jnp.bfloat16), we_ref[...],
                     preferred_element_type=jnp.float32) + be_ref[...]
        inter = zc * g_ref[:, j, :] * pe
        z = jnp.dot(inter.astype(jnp.bfloat16), wgm_ref[...],
                    preferred_element_type=jnp.float32) + bgm_ref[...]
        gate = jax.nn.sigmoid(z[:, :d])
        mag = jax.nn.softplus(z[:, d:])
        acc = acc + gate * mag
    out_ref[...] = x_ref[...] + acc


def _interaction(atom_in_fea, zc, gathered3, nbr_fea, W_edge, b_edge,
                 W_gm, b_gm, block):
    n, d = atom_in_fea.shape
    m = nbr_fea.shape[1]
    de = nbr_fea.shape[2]
    grid = (n // block,)
    full = lambda i: (0, 0)
    return pl.pallas_call(
        _inter_body,
        grid=grid,
        in_specs=[
            pl.BlockSpec((block, d), lambda i: (i, 0)),
            pl.BlockSpec((block, d), lambda i: (i, 0)),
            pl.BlockSpec((block, m, d), lambda i: (i, 0, 0)),
            pl.BlockSpec((block, m, de), lambda i: (i, 0, 0)),
            pl.BlockSpec((de, d), full),
            pl.BlockSpec((1, d), full),
            pl.BlockSpec((d, 2 * d), full),
            pl.BlockSpec((1, 2 * d), full),
        ],
        out_specs=pl.BlockSpec((block, d), lambda i: (i, 0)),
        out_shape=jax.ShapeDtypeStruct((n, d), jnp.float32),
    )(atom_in_fea, zc, gathered3, nbr_fea, W_edge.astype(jnp.bfloat16),
      b_edge.reshape(1, d), W_gm, b_gm)


def kernel(atom_in_fea, nbr_fea, nbr_fea_idx, ln_scale, ln_bias, W_center,
           b_center, W_nbr, b_nbr, W_edge, b_edge, W_gate, b_gate, W_mag,
           b_mag):
    n, d = atom_in_fea.shape
    m = nbr_fea_idx.shape[1]
    zc, zn = _projections(atom_in_fea, ln_scale, ln_bias, W_center, b_center,
                          W_nbr, b_nbr, block=400)
    gathered = _sc_gather(zn, nbr_fea_idx.reshape(-1), window=128)
    gathered3 = gathered.reshape(n, m, d)
    W_gm = jnp.concatenate([W_gate, W_mag], axis=1).astype(jnp.bfloat16)
    b_gm = jnp.concatenate([b_gate, b_mag]).reshape(1, 2 * d)
    return _interaction(atom_in_fea, zc, gathered3, nbr_fea, W_edge, b_edge,
                        W_gm, b_gm, block=200)
